# baseline (device time: 12349 ns/iter reference)
import jax
import jax.numpy as jnp
from jax import lax
from jax.experimental import pallas as pl
from jax.experimental.pallas import tpu as pltpu

N_DEV = 4
ROW_CHUNKS = 8


def kernel(x):
    m_per, n = x.shape
    m_global = m_per * N_DEV
    m_blk = m_per // ROW_CHUNKS

    def body(x_ref, out_ref, acc_ref, own_ref, comm_ref, send_sems, recv_sems):
        i = pl.program_id(0)
        my_pos = lax.axis_index("i")

        @pl.when(i == 0)
        def _():
            barrier_sem = pltpu.get_barrier_semaphore()
            for off in range(1, N_DEV):
                pl.semaphore_signal(
                    barrier_sem, inc=1,
                    device_id=((my_pos + off) % N_DEV,),
                    device_id_type=pl.DeviceIdType.MESH,
                )
            pl.semaphore_wait(barrier_sem, N_DEV - 1)

        blk = jnp.sum(x_ref[:, :], axis=0, keepdims=True)

        @pl.when(i == 0)
        def _():
            acc_ref[:, :] = blk

        @pl.when(i > 0)
        def _():
            acc_ref[:, :] = acc_ref[:, :] + blk

        @pl.when(i == ROW_CHUNKS - 1)
        def _():
            own_ref[:, :] = acc_ref[:, :]

            sends = []
            for off in range(1, N_DEV):
                rdma = pltpu.make_async_remote_copy(
                    src_ref=own_ref,
                    dst_ref=comm_ref.at[3 - off],
                    send_sem=send_sems.at[off - 1],
                    recv_sem=recv_sems.at[3 - off],
                    device_id=((my_pos + off) % N_DEV,),
                    device_id_type=pl.DeviceIdType.MESH,
                )
                rdma.start()
                sends.append(rdma)

            acc = own_ref[:, :]
            for off in range(1, N_DEV):
                slot = off - 1
                recv = pltpu.make_async_remote_copy(
                    src_ref=own_ref,
                    dst_ref=comm_ref.at[slot],
                    send_sem=send_sems.at[off - 1],
                    recv_sem=recv_sems.at[slot],
                    device_id=((my_pos + off) % N_DEV,),
                    device_id_type=pl.DeviceIdType.MESH,
                )
                recv.wait_recv()
                acc = acc + comm_ref[slot, :, :]

            out_ref[:, :] = acc * (1.0 / m_global)

            for rdma in sends:
                rdma.wait_send()

    return pl.pallas_call(
        body,
        grid=(ROW_CHUNKS,),
        out_shape=jax.ShapeDtypeStruct((1, n), jnp.float32),
        in_specs=[
            pl.BlockSpec((m_blk, n), lambda i: (i, 0), memory_space=pltpu.VMEM)
        ],
        out_specs=pl.BlockSpec((1, n), lambda i: (0, 0), memory_space=pltpu.VMEM),
        scratch_shapes=[
            pltpu.VMEM((1, n), jnp.float32),
            pltpu.VMEM((1, n), jnp.float32),
            pltpu.VMEM((N_DEV - 1, 1, n), jnp.float32),
            pltpu.SemaphoreType.DMA((N_DEV - 1,)),
            pltpu.SemaphoreType.DMA((N_DEV - 1,)),
        ],
        compiler_params=pltpu.CompilerParams(collective_id=0),
    )(x)


# device time: 11971 ns/iter; 1.0316x vs baseline; 1.0316x over previous
import jax
import jax.numpy as jnp
from jax import lax
from jax.experimental import pallas as pl
from jax.experimental.pallas import tpu as pltpu

N_DEV = 4
ROW_CHUNKS = 8


def kernel(x):
    m_per, n = x.shape
    m_global = m_per * N_DEV
    m_blk = m_per // ROW_CHUNKS

    def body(x_ref, out_ref, acc_ref, own_ref, comm_ref, send_sems, recv_sems):
        i = pl.program_id(0)
        my_pos = lax.axis_index("i")

        @pl.when(i == 0)
        def _():
            barrier_sem = pltpu.get_barrier_semaphore()
            for off in range(1, N_DEV):
                pl.semaphore_signal(
                    barrier_sem, inc=1,
                    device_id=((my_pos + off) % N_DEV,),
                    device_id_type=pl.DeviceIdType.MESH,
                )
            pl.semaphore_wait(barrier_sem, N_DEV - 1)

        blk = jnp.sum(x_ref[:, :], axis=0, keepdims=True)

        @pl.when(i == 0)
        def _():
            acc_ref[:, :] = blk

        @pl.when(i > 0)
        def _():
            acc_ref[:, :] = acc_ref[:, :] + blk

        @pl.when(i == ROW_CHUNKS - 1)
        def _():
            own_ref[:, :] = acc_ref[:, :]

            sends = []
            for off in range(1, N_DEV):
                rdma = pltpu.make_async_remote_copy(
                    src_ref=own_ref,
                    dst_ref=comm_ref.at[3 - off],
                    send_sem=send_sems.at[off - 1],
                    recv_sem=recv_sems.at[3 - off],
                    device_id=((my_pos + off) % N_DEV,),
                    device_id_type=pl.DeviceIdType.MESH,
                )
                rdma.start()
                sends.append(rdma)

            acc = own_ref[:, :]
            for off in range(1, N_DEV):
                slot = off - 1
                recv = pltpu.make_async_remote_copy(
                    src_ref=own_ref,
                    dst_ref=comm_ref.at[slot],
                    send_sem=send_sems.at[off - 1],
                    recv_sem=recv_sems.at[slot],
                    device_id=((my_pos + off) % N_DEV,),
                    device_id_type=pl.DeviceIdType.MESH,
                )
                recv.wait_recv()
                acc = acc + comm_ref[slot, :, :]

            out_ref[:, :] = acc * (1.0 / m_global)

            for rdma in sends:
                rdma.wait_send()

    return pl.pallas_call(
        body,
        grid=(ROW_CHUNKS,),
        out_shape=jax.ShapeDtypeStruct((1, n), jnp.float32),
        in_specs=[pl.BlockSpec((m_blk, n), lambda i: (i, 0))],
        out_specs=pl.BlockSpec((1, n), lambda i: (0, 0)),
        scratch_shapes=[
            pltpu.VMEM((1, n), jnp.float32),
            pltpu.VMEM((1, n), jnp.float32),
            pltpu.VMEM((N_DEV - 1, 1, n), jnp.float32),
            pltpu.SemaphoreType.DMA((N_DEV - 1,)),
            pltpu.SemaphoreType.DMA((N_DEV - 1,)),
        ],
        compiler_params=pltpu.CompilerParams(collective_id=0),
    )(x)
